# Initial kernel scaffold; baseline (speedup 1.0000x reference)
import jax
import jax.numpy as jnp
from jax.experimental import pallas as pl


def kernel(x, onehots, edge_index, batch_sample_indices, n_sample_nodes, adjs, conv1_w, conv1_b, conv2_w, conv2_b, lin16_w, lin16_b, W1, b1, bn_gamma, bn_beta, W2, b2):
    raise NotImplementedError("write your pallas kernel here")


# speedup vs baseline: 5.2131x; 5.2131x over previous
import functools

import jax
import jax.numpy as jnp
from jax import lax
from jax.experimental import pallas as pl
from jax.experimental.pallas import tpu as pltpu
from jax.experimental.pallas import tpu_sc as plsc

_W = 80


def _sc_aggregate(feat2, idx2, recv, n):
    dc = feat2.shape[1]
    e = idx2.shape[1]
    epw = e // 16
    nwin = epw // _W
    rps = 640
    npad = 16 * rps
    zr = 128
    nz = rps // zr
    last_rows = n - 15 * rps

    mesh = plsc.VectorSubcoreMesh(core_axis_name="c", subcore_axis_name="s")

    @functools.partial(
        pl.kernel,
        out_type=jax.ShapeDtypeStruct((2, n, dc), jnp.float32),
        mesh=mesh,
        scratch_types=[
            pltpu.VMEM((epw,), jnp.int32),
            pltpu.VMEM((_W,), jnp.int32),
            pltpu.VMEM((_W,), jnp.int32),
            pltpu.VMEM((_W, dc), jnp.float32),
            pltpu.VMEM((_W, dc), jnp.float32),
            pltpu.VMEM((zr, dc), jnp.float32),
            pltpu.VMEM_SHARED((npad, dc), jnp.float32),
            pltpu.SemaphoreType.DMA,
            pltpu.SemaphoreType.DMA,
        ],
        compiler_params=pltpu.CompilerParams(use_tc_tiling_on_sc=False),
    )
    def agg_kernel(feat_hbm, send_hbm, recv_hbm, out_hbm,
                   sall, ridx0, ridx1, rows0, rows1, zbuf, acc, sem0, sem1):
        cid = lax.axis_index("c")
        sid = lax.axis_index("s")

        @pl.loop(0, zr)
        def _(r):
            @pl.loop(0, dc, step=16)
            def _(c0):
                zbuf[r, pl.ds(c0, 16)] = jnp.zeros((16,), jnp.float32)

        @pl.loop(0, nz)
        def _(j):
            pltpu.sync_copy(zbuf, acc.at[pl.ds(sid * rps + j * zr, zr)])

        plsc.subcore_barrier()

        base = sid * epw
        pltpu.sync_copy(send_hbm.at[cid, pl.ds(base, epw)], sall)

        def issue(w, ridx, rows, sem):
            pltpu.sync_copy(recv_hbm.at[pl.ds(base + w * _W, _W)], ridx)
            pltpu.async_copy(feat_hbm.at[sall.at[pl.ds(w * _W, _W)]], rows, sem)

        def finish(w, ridx, rows, sem):
            pltpu.make_async_copy(
                feat_hbm.at[sall.at[pl.ds(w * _W, _W)]], rows, sem).wait()
            pltpu.sync_copy(rows, acc.at[ridx], add=True)

        assert nwin % 2 == 0
        issue(0, ridx0, rows0, sem0)

        @pl.loop(0, nwin - 2, step=2)
        def _(w):
            issue(w + 1, ridx1, rows1, sem1)
            finish(w, ridx0, rows0, sem0)
            issue(w + 2, ridx0, rows0, sem0)
            finish(w + 1, ridx1, rows1, sem1)

        issue(nwin - 1, ridx1, rows1, sem1)
        finish(nwin - 2, ridx0, rows0, sem0)
        finish(nwin - 1, ridx1, rows1, sem1)

        plsc.subcore_barrier()

        @pl.when(sid < 15)
        def _():
            pltpu.sync_copy(acc.at[pl.ds(sid * rps, rps)],
                            out_hbm.at[cid, pl.ds(sid * rps, rps)])

        @pl.when(sid == 15)
        def _():
            pltpu.sync_copy(acc.at[pl.ds(15 * rps, last_rows)],
                            out_hbm.at[cid, pl.ds(15 * rps, last_rows)])

    return agg_kernel(feat2, idx2, recv)


def _dense_body(part_ref, oh_ref, w1k0_ref, w1k1_ref, w1k2_ref, cb1_ref,
                w2m_ref, cb2_ref, lwT_ref, lb_ref, W1Ta_ref, W1Tb_ref,
                b1r_ref, gam_ref, bet_ref, W2T_ref, b2r_ref,
                out_h_ref, out_oh_ref, h_scr, st_scr, *, bsz, n, dx, l):
    p = pl.program_id(0)
    j = pl.program_id(1)
    hi = lax.Precision.HIGHEST

    agg_x = jnp.concatenate([part_ref[0], part_ref[1][:, :dx - 80]], axis=1)
    new_oh = part_ref[1][:, dx - 80:dx - 80 + l] + oh_ref[...]
    out_oh_ref[...] = new_oh

    @pl.when(p == 0)
    def _phase0():
        v = new_oh
        vi = v[:, :, None]
        vj = v[:, None, :]
        ii = lax.broadcasted_iota(jnp.int32, (1, l, l), 1)
        jj = lax.broadcasted_iota(jnp.int32, (1, l, l), 2)
        less = (vj < vi) | ((vj == vi) & (jj < ii))
        rank = jnp.sum(less.astype(jnp.int32), axis=2)
        rr = lax.broadcasted_iota(jnp.int32, (1, 1, l), 2)
        onehot = (rank[:, :, None] == rr).astype(jnp.float32)
        s = jnp.sum(vi * onehot, axis=1)

        zc = jnp.zeros((bsz, 1), jnp.float32)
        s_l = jnp.concatenate([zc, s[:, :-1]], axis=1)
        s_r = jnp.concatenate([s[:, 1:], zc], axis=1)
        k0 = w1k0_ref[...].reshape(1, 1, 8)
        k1 = w1k1_ref[...].reshape(1, 1, 8)
        k2 = w1k2_ref[...].reshape(1, 1, 8)
        cb1 = cb1_ref[...].reshape(1, 1, 8)
        h1 = jnp.maximum(
            s_l[:, :, None] * k0 + s[:, :, None] * k1 + s_r[:, :, None] * k2
            + cb1, 0.0)
        z3 = jnp.zeros((bsz, 1, 8), jnp.float32)
        h1l = jnp.concatenate([z3, h1[:, :-1, :]], axis=1)
        h1r = jnp.concatenate([h1[:, 1:, :], z3], axis=1)
        hc = jnp.concatenate([h1l, h1, h1r], axis=2)
        h2 = lax.dot_general(hc, w2m_ref[...], (((2,), (0,)), ((), ())),
                             precision=hi,
                             preferred_element_type=jnp.float32)
        h2 = jnp.maximum(h2 + cb2_ref[...].reshape(1, 1, 16), 0.0)
        h2m = jnp.mean(h2, axis=1)
        res = jnp.dot(h2m, lwT_ref[...], precision=hi,
                      preferred_element_type=jnp.float32) + lb_ref[...]

        hb = (jnp.dot(agg_x, W1Ta_ref[...], precision=hi,
                      preferred_element_type=jnp.float32)
              + jnp.dot(res, W1Tb_ref[...], precision=hi,
                        preferred_element_type=jnp.float32)
              + b1r_ref[...])
        h_scr[pl.ds(j * bsz, bsz), :] = hb

        @pl.when(j == 0)
        def _():
            st_scr[...] = jnp.zeros_like(st_scr)

        st_scr[0:1, :] += jnp.sum(hb, axis=0, keepdims=True)
        st_scr[1:2, :] += jnp.sum(hb * hb, axis=0, keepdims=True)
        out_h_ref[...] = hb

    @pl.when(p == 1)
    def _phase1():
        mu = st_scr[0:1, :] / n
        var = st_scr[1:2, :] / n - mu * mu
        rs = lax.rsqrt(var + 1e-5)
        hb = h_scr[pl.ds(j * bsz, bsz), :]
        hn = jnp.maximum((hb - mu) * rs * gam_ref[...] + bet_ref[...], 0.0)
        out_h_ref[...] = (jnp.dot(hn, W2T_ref[...], precision=hi,
                                  preferred_element_type=jnp.float32)
                          + b2r_ref[...])


def _dense_call(partial, onehots, w1k0, w1k1, w1k2, cb1, w2m, cb2, lwT, lb,
                W1Ta, W1Tb, b1r, gam, bet, W2T, b2r, interpret=False):
    n, l = onehots.shape
    dx = 128
    dc = partial.shape[2]
    bsz = 400
    nb = n // bsz

    def full(a):
        return pl.BlockSpec(a.shape, lambda p, j: (0,) * a.ndim)

    body = functools.partial(_dense_body, bsz=bsz, n=n, dx=dx, l=l)
    return pl.pallas_call(
        body,
        grid=(2, nb),
        in_specs=[
            pl.BlockSpec((2, bsz, dc), lambda p, j: (0, j, 0)),
            pl.BlockSpec((bsz, l), lambda p, j: (j, 0)),
            full(w1k0), full(w1k1), full(w1k2), full(cb1), full(w2m),
            full(cb2), full(lwT), full(lb), full(W1Ta), full(W1Tb),
            full(b1r), full(gam), full(bet), full(W2T), full(b2r),
        ],
        out_specs=[
            pl.BlockSpec((bsz, dx), lambda p, j: (p * j, 0)),
            pl.BlockSpec((bsz, l), lambda p, j: (j, 0)),
        ],
        out_shape=[
            jax.ShapeDtypeStruct((n, dx), jnp.float32),
            jax.ShapeDtypeStruct((n, l), jnp.float32),
        ],
        scratch_shapes=[
            pltpu.VMEM((n, dx), jnp.float32),
            pltpu.VMEM((8, dx), jnp.float32),
        ],
        compiler_params=pltpu.CompilerParams(
            dimension_semantics=("arbitrary", "arbitrary")),
        interpret=interpret,
    )(partial, onehots, w1k0, w1k1, w1k2, cb1, w2m, cb2, lwT, lb,
      W1Ta, W1Tb, b1r, gam, bet, W2T, b2r)


def kernel(x, onehots, edge_index, batch_sample_indices, n_sample_nodes, adjs,
           conv1_w, conv1_b, conv2_w, conv2_b, lin16_w, lin16_b,
           W1, b1, bn_gamma, bn_beta, W2, b2):
    n, dx = x.shape
    l = onehots.shape[1]

    fa = x[:, :80]
    fb = jnp.concatenate(
        [x[:, 80:], onehots, jnp.zeros((n, 160 - dx - l), jnp.float32)],
        axis=1)
    feat2 = jnp.concatenate([fa, fb], axis=0)
    send = edge_index[0]
    recv = edge_index[1]
    idx2 = jnp.stack([send, send + n])
    partial = _sc_aggregate(feat2, idx2, recv, n)

    w1k0 = conv1_w[:, 0, 0].reshape(1, 8)
    w1k1 = conv1_w[:, 0, 1].reshape(1, 8)
    w1k2 = conv1_w[:, 0, 2].reshape(1, 8)
    cb1 = conv1_b.reshape(1, 8)
    w2m = jnp.transpose(conv2_w, (2, 1, 0)).reshape(24, 16)
    cb2 = conv2_b.reshape(1, 16)
    lwT = lin16_w.T
    lb = lin16_b.reshape(1, 8)
    W1Ta = W1[:, :dx].T
    W1Tb = W1[:, dx:].T
    b1r = b1.reshape(1, dx)
    gam = bn_gamma.reshape(1, dx)
    bet = bn_beta.reshape(1, dx)
    W2T = W2.T
    b2r = b2.reshape(1, dx)

    h, new_oh = _dense_call(partial, onehots, w1k0, w1k1, w1k2, cb1, w2m, cb2,
                            lwT, lb, W1Ta, W1Tb, b1r, gam, bet, W2T, b2r)
    return (h, new_oh)


# speedup vs baseline: 7.2940x; 1.3992x over previous
import functools

import jax
import jax.numpy as jnp
from jax import lax
from jax.experimental import pallas as pl
from jax.experimental.pallas import tpu as pltpu
from jax.experimental.pallas import tpu_sc as plsc

_W = 80


def _sc_aggregate(feat2, idx2, recv, n):
    dc = feat2.shape[1]
    e = idx2.shape[1]
    epw = e // 16
    nwin = epw // _W
    rps = 640
    npad = 16 * rps
    zr = 128
    nz = rps // zr
    last_rows = n - 15 * rps

    mesh = plsc.VectorSubcoreMesh(core_axis_name="c", subcore_axis_name="s")

    @functools.partial(
        pl.kernel,
        out_type=jax.ShapeDtypeStruct((2, n, dc), jnp.float32),
        mesh=mesh,
        scratch_types=[
            pltpu.VMEM((epw,), jnp.int32),
            pltpu.VMEM((_W,), jnp.int32),
            pltpu.VMEM((_W,), jnp.int32),
            pltpu.VMEM((_W, dc), jnp.float32),
            pltpu.VMEM((_W, dc), jnp.float32),
            pltpu.VMEM((zr, dc), jnp.float32),
            pltpu.VMEM_SHARED((npad, dc), jnp.float32),
            pltpu.SemaphoreType.DMA,
            pltpu.SemaphoreType.DMA,
        ],
        compiler_params=pltpu.CompilerParams(use_tc_tiling_on_sc=False),
    )
    def agg_kernel(feat_hbm, send_hbm, recv_hbm, out_hbm,
                   sall, ridx0, ridx1, rows0, rows1, zbuf, acc, sem0, sem1):
        cid = lax.axis_index("c")
        sid = lax.axis_index("s")

        @pl.loop(0, zr)
        def _(r):
            @pl.loop(0, dc, step=16)
            def _(c0):
                zbuf[r, pl.ds(c0, 16)] = jnp.zeros((16,), jnp.float32)

        @pl.loop(0, nz)
        def _(j):
            pltpu.sync_copy(zbuf, acc.at[pl.ds(sid * rps + j * zr, zr)])

        plsc.subcore_barrier()

        base = sid * epw
        pltpu.sync_copy(send_hbm.at[cid, pl.ds(base, epw)], sall)

        def issue(w, ridx, rows, sem):
            pltpu.sync_copy(recv_hbm.at[pl.ds(base + w * _W, _W)], ridx)
            pltpu.async_copy(feat_hbm.at[sall.at[pl.ds(w * _W, _W)]], rows, sem)

        def finish(w, ridx, rows, sem):
            pltpu.make_async_copy(
                feat_hbm.at[sall.at[pl.ds(w * _W, _W)]], rows, sem).wait()
            pltpu.sync_copy(rows, acc.at[ridx], add=True)

        assert nwin % 2 == 0
        issue(0, ridx0, rows0, sem0)

        @pl.loop(0, nwin - 2, step=2)
        def _(w):
            issue(w + 1, ridx1, rows1, sem1)
            finish(w, ridx0, rows0, sem0)
            issue(w + 2, ridx0, rows0, sem0)
            finish(w + 1, ridx1, rows1, sem1)

        issue(nwin - 1, ridx1, rows1, sem1)
        finish(nwin - 2, ridx0, rows0, sem0)
        finish(nwin - 1, ridx1, rows1, sem1)

        plsc.subcore_barrier()

        @pl.when(sid < 15)
        def _():
            pltpu.sync_copy(acc.at[pl.ds(sid * rps, rps)],
                            out_hbm.at[cid, pl.ds(sid * rps, rps)])

        @pl.when(sid == 15)
        def _():
            pltpu.sync_copy(acc.at[pl.ds(15 * rps, last_rows)],
                            out_hbm.at[cid, pl.ds(15 * rps, last_rows)])

    return agg_kernel(feat2, idx2, recv)


_BITONIC_STAGES = [(2, 1), (4, 2), (4, 1), (8, 4), (8, 2), (8, 1),
                   (16, 8), (16, 4), (16, 2), (16, 1)]


def _dense_body(part_ref, oh_ref, A1_ref, cb1t_ref, A2_ref, cb2t_ref,
                A3_ref, lb_ref, W1Ta_ref, W1Tb_ref,
                b1r_ref, gam_ref, bet_ref, W2T_ref, b2r_ref,
                out_h_ref, out_oh_ref, h_scr, st_scr, *, bsz, n, dx, l):
    p = pl.program_id(0)
    j = pl.program_id(1)
    hi = lax.Precision.HIGHEST

    agg_x = jnp.concatenate([part_ref[0], part_ref[1][:, :dx - 80]], axis=1)
    new_oh = part_ref[1][:, dx - 80:dx - 80 + l] + oh_ref[...]
    out_oh_ref[...] = new_oh

    @pl.when(p == 0)
    def _phase0():
        vp = new_oh
        li = lax.broadcasted_iota(jnp.int32, (1, l), 1)
        for k, jj in _BITONIC_STAGES:
            zc = jnp.zeros((bsz, jj), jnp.float32)
            up = jnp.concatenate([vp[:, jj:], zc], axis=1)
            dn = jnp.concatenate([zc, vp[:, :l - jj]], axis=1)
            is_lo = (li & jj) == 0
            pv = jnp.where(is_lo, up, dn)
            keep_min = is_lo == ((li & k) == 0)
            vp = jnp.where(keep_min, jnp.minimum(vp, pv),
                           jnp.maximum(vp, pv))
        s = vp

        h1f = jnp.maximum(
            jnp.dot(s, A1_ref[...], precision=hi,
                    preferred_element_type=jnp.float32) + cb1t_ref[...], 0.0)
        h2f = jnp.maximum(
            jnp.dot(h1f, A2_ref[...], precision=hi,
                    preferred_element_type=jnp.float32) + cb2t_ref[...], 0.0)
        res = jnp.dot(h2f, A3_ref[...], precision=hi,
                      preferred_element_type=jnp.float32) + lb_ref[...]

        hb = (jnp.dot(agg_x, W1Ta_ref[...], precision=hi,
                      preferred_element_type=jnp.float32)
              + jnp.dot(res, W1Tb_ref[...], precision=hi,
                        preferred_element_type=jnp.float32)
              + b1r_ref[...])
        h_scr[pl.ds(j * bsz, bsz), :] = hb

        @pl.when(j == 0)
        def _():
            st_scr[...] = jnp.zeros_like(st_scr)

        st_scr[0:1, :] += jnp.sum(hb, axis=0, keepdims=True)
        st_scr[1:2, :] += jnp.sum(hb * hb, axis=0, keepdims=True)
        out_h_ref[...] = hb

    @pl.when(p == 1)
    def _phase1():
        mu = st_scr[0:1, :] / n
        var = st_scr[1:2, :] / n - mu * mu
        rs = lax.rsqrt(var + 1e-5)
        hb = h_scr[pl.ds(j * bsz, bsz), :]
        hn = jnp.maximum((hb - mu) * rs * gam_ref[...] + bet_ref[...], 0.0)
        out_h_ref[...] = (jnp.dot(hn, W2T_ref[...], precision=hi,
                                  preferred_element_type=jnp.float32)
                          + b2r_ref[...])


def _dense_call(partial, onehots, A1, cb1t, A2, cb2t, A3, lb,
                W1Ta, W1Tb, b1r, gam, bet, W2T, b2r, interpret=False):
    n, l = onehots.shape
    dx = 128
    dc = partial.shape[2]
    bsz = 400
    nb = n // bsz

    def full(a):
        return pl.BlockSpec(a.shape, lambda p, j: (0,) * a.ndim)

    body = functools.partial(_dense_body, bsz=bsz, n=n, dx=dx, l=l)
    return pl.pallas_call(
        body,
        grid=(2, nb),
        in_specs=[
            pl.BlockSpec((2, bsz, dc), lambda p, j: (0, j, 0)),
            pl.BlockSpec((bsz, l), lambda p, j: (j, 0)),
            full(A1), full(cb1t), full(A2), full(cb2t), full(A3),
            full(lb), full(W1Ta), full(W1Tb),
            full(b1r), full(gam), full(bet), full(W2T), full(b2r),
        ],
        out_specs=[
            pl.BlockSpec((bsz, dx), lambda p, j: (p * j, 0)),
            pl.BlockSpec((bsz, l), lambda p, j: (j, 0)),
        ],
        out_shape=[
            jax.ShapeDtypeStruct((n, dx), jnp.float32),
            jax.ShapeDtypeStruct((n, l), jnp.float32),
        ],
        scratch_shapes=[
            pltpu.VMEM((n, dx), jnp.float32),
            pltpu.VMEM((8, dx), jnp.float32),
        ],
        compiler_params=pltpu.CompilerParams(
            dimension_semantics=("arbitrary", "arbitrary")),
        interpret=interpret,
    )(partial, onehots, A1, cb1t, A2, cb2t, A3, lb,
      W1Ta, W1Tb, b1r, gam, bet, W2T, b2r)


def kernel(x, onehots, edge_index, batch_sample_indices, n_sample_nodes, adjs,
           conv1_w, conv1_b, conv2_w, conv2_b, lin16_w, lin16_b,
           W1, b1, bn_gamma, bn_beta, W2, b2):
    n, dx = x.shape
    l = onehots.shape[1]

    fa = x[:, :80]
    fb = jnp.concatenate(
        [x[:, 80:], onehots, jnp.zeros((n, 160 - dx - l), jnp.float32)],
        axis=1)
    feat2 = jnp.concatenate([fa, fb], axis=0)
    send = edge_index[0]
    recv = edge_index[1]
    idx2 = jnp.stack([send, send + n])
    partial = _sc_aggregate(feat2, idx2, recv, n)

    eyes = [jnp.eye(l, k=1 - k, dtype=jnp.float32) for k in range(3)]
    A1 = sum(eyes[k][:, :, None] * conv1_w[:, 0, k][None, None, :]
             for k in range(3)).reshape(l, l * 8)
    cb1t = jnp.tile(conv1_b, (l,)).reshape(1, l * 8)
    A2 = sum(eyes[k][:, None, :, None]
             * jnp.transpose(conv2_w[:, :, k])[None, :, None, :]
             for k in range(3)).reshape(l * 8, l * 16)
    cb2t = jnp.tile(conv2_b, (l,)).reshape(1, l * 16)
    A3 = jnp.tile(lin16_w.T / l, (l, 1))
    lb = lin16_b.reshape(1, 8)
    W1Ta = W1[:, :dx].T
    W1Tb = W1[:, dx:].T
    b1r = b1.reshape(1, dx)
    gam = bn_gamma.reshape(1, dx)
    bet = bn_beta.reshape(1, dx)
    W2T = W2.T
    b2r = b2.reshape(1, dx)

    h, new_oh = _dense_call(partial, onehots, A1, cb1t, A2, cb2t, A3, lb,
                            W1Ta, W1Tb, b1r, gam, bet, W2T, b2r)
    return (h, new_oh)


# speedup vs baseline: 9.2366x; 1.2663x over previous
import functools

import jax
import jax.numpy as jnp
from jax import lax
from jax.experimental import pallas as pl
from jax.experimental.pallas import tpu as pltpu
from jax.experimental.pallas import tpu_sc as plsc

_W = 80
_NBUF = 2


def _sc_aggregate(feat2, idx2, recv, n):
    dc = feat2.shape[1]
    e = idx2.shape[1]
    epw = e // 16
    nwin = epw // _W
    rps = 640
    npad = 16 * rps
    zr = 128
    nz = rps // zr
    last_rows = n - 15 * rps

    mesh = plsc.VectorSubcoreMesh(core_axis_name="c", subcore_axis_name="s")

    @functools.partial(
        pl.kernel,
        out_type=jax.ShapeDtypeStruct((2, n, dc), jnp.float32),
        mesh=mesh,
        scratch_types=[
            pltpu.VMEM((epw,), jnp.int32),
            pltpu.VMEM((nwin, _W), jnp.int32),
            pltpu.VMEM((2 * _NBUF, _W, dc), jnp.float32),
            pltpu.VMEM((zr, dc), jnp.float32),
            pltpu.VMEM_SHARED((npad, dc), jnp.float32),
        ] + [pltpu.SemaphoreType.DMA] * 4,
        compiler_params=pltpu.CompilerParams(use_tc_tiling_on_sc=False),
    )
    def agg_kernel(feat_hbm, send_hbm, recv_hbm, out_hbm,
                   sall, rall, rows, zbuf, acc, gsem0, gsem1, ssem0, ssem1):
        gsem = (gsem0, gsem1)
        ssem = (ssem0, ssem1)
        cid = lax.axis_index("c")
        sid = lax.axis_index("s")

        @pl.loop(0, zr)
        def _(r):
            @pl.loop(0, dc, step=16)
            def _(c0):
                zbuf[r, pl.ds(c0, 16)] = jnp.zeros((16,), jnp.float32)

        @pl.loop(0, nz)
        def _(j):
            pltpu.sync_copy(zbuf, acc.at[pl.ds(sid * rps + j * zr, zr)])

        plsc.subcore_barrier()

        base = sid * epw
        pltpu.sync_copy(send_hbm.at[cid, pl.ds(base, epw)], sall)
        pltpu.sync_copy(recv_hbm.at[sid], rall)

        def issue_group(a, q):
            for b in range(_NBUF):
                w = q * _NBUF + b
                pltpu.async_copy(
                    feat_hbm.at[sall.at[pl.ds(w * _W, _W)]],
                    rows.at[a * _NBUF + b], gsem[a])

        def process_group(a, q):
            for b in range(_NBUF):
                w = q * _NBUF + b
                pltpu.make_async_copy(
                    feat_hbm.at[sall.at[pl.ds(w * _W, _W)]],
                    rows.at[a * _NBUF + b], gsem[a]).wait()
            for b in range(_NBUF):
                w = q * _NBUF + b
                pltpu.async_copy(rows.at[a * _NBUF + b], acc.at[rall.at[w]],
                                 ssem[a], add=True)
            for b in range(_NBUF):
                w = q * _NBUF + b
                pltpu.make_async_copy(rows.at[a * _NBUF + b],
                                      acc.at[rall.at[w]], ssem[a]).wait()

        nq = nwin // _NBUF
        assert nq % 2 == 1
        issue_group(0, 0)

        @pl.loop(0, nq - 1, step=2)
        def _(q):
            issue_group(1, q + 1)
            process_group(0, q)
            issue_group(0, q + 2)
            process_group(1, q + 1)

        process_group(0, nq - 1)

        plsc.subcore_barrier()

        @pl.when(sid < 15)
        def _():
            pltpu.sync_copy(acc.at[pl.ds(sid * rps, rps)],
                            out_hbm.at[cid, pl.ds(sid * rps, rps)])

        @pl.when(sid == 15)
        def _():
            pltpu.sync_copy(acc.at[pl.ds(15 * rps, last_rows)],
                            out_hbm.at[cid, pl.ds(15 * rps, last_rows)])

    return agg_kernel(feat2, idx2, recv.reshape(16, nwin, _W))


_BITONIC_STAGES = [(2, 1), (4, 2), (4, 1), (8, 4), (8, 2), (8, 1),
                   (16, 8), (16, 4), (16, 2), (16, 1)]


def _dense_body(part_ref, oh_ref, A1_ref, cb1t_ref, A2_ref, cb2t_ref,
                A3_ref, lb_ref, W1Ta_ref, W1Tb_ref,
                b1r_ref, gam_ref, bet_ref, W2T_ref, b2r_ref,
                out_h_ref, out_oh_ref, h_scr, st_scr, *, bsz, n, dx, l):
    p = pl.program_id(0)
    j = pl.program_id(1)
    hi = lax.Precision.HIGHEST

    agg_x = jnp.concatenate([part_ref[0], part_ref[1][:, :dx - 80]], axis=1)
    new_oh = part_ref[1][:, dx - 80:dx - 80 + l] + oh_ref[...]
    out_oh_ref[...] = new_oh

    @pl.when(p == 0)
    def _phase0():
        vp = new_oh
        li = lax.broadcasted_iota(jnp.int32, (1, l), 1)
        for k, jj in _BITONIC_STAGES:
            zc = jnp.zeros((bsz, jj), jnp.float32)
            up = jnp.concatenate([vp[:, jj:], zc], axis=1)
            dn = jnp.concatenate([zc, vp[:, :l - jj]], axis=1)
            is_lo = (li & jj) == 0
            pv = jnp.where(is_lo, up, dn)
            keep_min = is_lo == ((li & k) == 0)
            vp = jnp.where(keep_min, jnp.minimum(vp, pv),
                           jnp.maximum(vp, pv))
        s = vp

        h1f = jnp.maximum(
            jnp.dot(s, A1_ref[...], precision=hi,
                    preferred_element_type=jnp.float32) + cb1t_ref[...], 0.0)
        h2f = jnp.maximum(
            jnp.dot(h1f, A2_ref[...], precision=hi,
                    preferred_element_type=jnp.float32) + cb2t_ref[...], 0.0)
        res = jnp.dot(h2f, A3_ref[...], precision=hi,
                      preferred_element_type=jnp.float32) + lb_ref[...]

        hb = (jnp.dot(agg_x, W1Ta_ref[...], precision=hi,
                      preferred_element_type=jnp.float32)
              + jnp.dot(res, W1Tb_ref[...], precision=hi,
                        preferred_element_type=jnp.float32)
              + b1r_ref[...])
        h_scr[pl.ds(j * bsz, bsz), :] = hb

        @pl.when(j == 0)
        def _():
            st_scr[...] = jnp.zeros_like(st_scr)

        st_scr[0:1, :] += jnp.sum(hb, axis=0, keepdims=True)
        st_scr[1:2, :] += jnp.sum(hb * hb, axis=0, keepdims=True)
        out_h_ref[...] = hb

    @pl.when(p == 1)
    def _phase1():
        mu = st_scr[0:1, :] / n
        var = st_scr[1:2, :] / n - mu * mu
        rs = lax.rsqrt(var + 1e-5)
        hb = h_scr[pl.ds(j * bsz, bsz), :]
        hn = jnp.maximum((hb - mu) * rs * gam_ref[...] + bet_ref[...], 0.0)
        out_h_ref[...] = (jnp.dot(hn, W2T_ref[...], precision=hi,
                                  preferred_element_type=jnp.float32)
                          + b2r_ref[...])


def _dense_call(partial, onehots, A1, cb1t, A2, cb2t, A3, lb,
                W1Ta, W1Tb, b1r, gam, bet, W2T, b2r, interpret=False):
    n, l = onehots.shape
    dx = 128
    dc = partial.shape[2]
    bsz = 400
    nb = n // bsz

    def full(a):
        return pl.BlockSpec(a.shape, lambda p, j: (0,) * a.ndim)

    body = functools.partial(_dense_body, bsz=bsz, n=n, dx=dx, l=l)
    return pl.pallas_call(
        body,
        grid=(2, nb),
        in_specs=[
            pl.BlockSpec((2, bsz, dc), lambda p, j: (0, j, 0)),
            pl.BlockSpec((bsz, l), lambda p, j: (j, 0)),
            full(A1), full(cb1t), full(A2), full(cb2t), full(A3),
            full(lb), full(W1Ta), full(W1Tb),
            full(b1r), full(gam), full(bet), full(W2T), full(b2r),
        ],
        out_specs=[
            pl.BlockSpec((bsz, dx), lambda p, j: (p * j, 0)),
            pl.BlockSpec((bsz, l), lambda p, j: (j, 0)),
        ],
        out_shape=[
            jax.ShapeDtypeStruct((n, dx), jnp.float32),
            jax.ShapeDtypeStruct((n, l), jnp.float32),
        ],
        scratch_shapes=[
            pltpu.VMEM((n, dx), jnp.float32),
            pltpu.VMEM((8, dx), jnp.float32),
        ],
        compiler_params=pltpu.CompilerParams(
            dimension_semantics=("arbitrary", "arbitrary")),
        interpret=interpret,
    )(partial, onehots, A1, cb1t, A2, cb2t, A3, lb,
      W1Ta, W1Tb, b1r, gam, bet, W2T, b2r)


def kernel(x, onehots, edge_index, batch_sample_indices, n_sample_nodes, adjs,
           conv1_w, conv1_b, conv2_w, conv2_b, lin16_w, lin16_b,
           W1, b1, bn_gamma, bn_beta, W2, b2):
    n, dx = x.shape
    l = onehots.shape[1]

    fa = x[:, :80]
    fb = jnp.concatenate(
        [x[:, 80:], onehots, jnp.zeros((n, 160 - dx - l), jnp.float32)],
        axis=1)
    feat2 = jnp.concatenate([fa, fb], axis=0)
    send = edge_index[0]
    recv = edge_index[1]
    idx2 = jnp.stack([send, send + n])
    partial = _sc_aggregate(feat2, idx2, recv, n)

    eyes = [jnp.eye(l, k=1 - k, dtype=jnp.float32) for k in range(3)]
    A1 = sum(eyes[k][:, :, None] * conv1_w[:, 0, k][None, None, :]
             for k in range(3)).reshape(l, l * 8)
    cb1t = jnp.tile(conv1_b, (l,)).reshape(1, l * 8)
    A2 = sum(eyes[k][:, None, :, None]
             * jnp.transpose(conv2_w[:, :, k])[None, :, None, :]
             for k in range(3)).reshape(l * 8, l * 16)
    cb2t = jnp.tile(conv2_b, (l,)).reshape(1, l * 16)
    A3 = jnp.tile(lin16_w.T / l, (l, 1))
    lb = lin16_b.reshape(1, 8)
    W1Ta = W1[:, :dx].T
    W1Tb = W1[:, dx:].T
    b1r = b1.reshape(1, dx)
    gam = bn_gamma.reshape(1, dx)
    bet = bn_beta.reshape(1, dx)
    W2T = W2.T
    b2r = b2.reshape(1, dx)

    h, new_oh = _dense_call(partial, onehots, A1, cb1t, A2, cb2t, A3, lb,
                            W1Ta, W1Tb, b1r, gam, bet, W2T, b2r)
    return (h, new_oh)


# speedup vs baseline: 9.4168x; 1.0195x over previous
import functools

import jax
import jax.numpy as jnp
from jax import lax
from jax.experimental import pallas as pl
from jax.experimental.pallas import tpu as pltpu
from jax.experimental.pallas import tpu_sc as plsc

_W = 80
_NBUF = 2


def _sc_aggregate(feat2, send, recv, n):
    dc = feat2.shape[1]
    e = send.shape[0]
    epw = e // 16
    nwin = epw // _W
    rps = 640
    npad = 16 * rps
    zr = 128
    nz = rps // zr
    last_rows = n - 15 * rps

    mesh = plsc.VectorSubcoreMesh(core_axis_name="c", subcore_axis_name="s")

    @functools.partial(
        pl.kernel,
        out_type=jax.ShapeDtypeStruct((2, n, dc), jnp.float32),
        mesh=mesh,
        scratch_types=[
            pltpu.VMEM((epw,), jnp.int32),
            pltpu.VMEM((nwin, _W), jnp.int32),
            pltpu.VMEM((2 * _NBUF, _W, dc), jnp.float32),
            pltpu.VMEM((zr, dc), jnp.float32),
            pltpu.VMEM_SHARED((npad, dc), jnp.float32),
        ] + [pltpu.SemaphoreType.DMA] * 4,
        compiler_params=pltpu.CompilerParams(use_tc_tiling_on_sc=False),
    )
    def agg_kernel(feat_hbm, send_hbm, recv_hbm, out_hbm,
                   sall, rall, rows, zbuf, acc, gsem0, gsem1, ssem0, ssem1):
        gsem = (gsem0, gsem1)
        ssem = (ssem0, ssem1)
        cid = lax.axis_index("c")
        sid = lax.axis_index("s")

        @pl.loop(0, zr)
        def _(r):
            @pl.loop(0, dc, step=16)
            def _(c0):
                zbuf[r, pl.ds(c0, 16)] = jnp.zeros((16,), jnp.float32)

        @pl.loop(0, nz)
        def _(j):
            pltpu.sync_copy(zbuf, acc.at[pl.ds(sid * rps + j * zr, zr)])

        plsc.subcore_barrier()

        base = sid * epw
        pltpu.sync_copy(send_hbm.at[pl.ds(base, epw)], sall)
        pltpu.sync_copy(recv_hbm.at[sid], rall)

        @pl.when(cid == 1)
        def _():
            @pl.loop(0, epw, step=16)
            def _(i):
                sall[pl.ds(i, 16)] = sall[pl.ds(i, 16)] + n

        def issue_group(a, q):
            for b in range(_NBUF):
                w = q * _NBUF + b
                pltpu.async_copy(
                    feat_hbm.at[sall.at[pl.ds(w * _W, _W)]],
                    rows.at[a * _NBUF + b], gsem[a])

        def process_group(a, q):
            for b in range(_NBUF):
                w = q * _NBUF + b
                pltpu.make_async_copy(
                    feat_hbm.at[sall.at[pl.ds(w * _W, _W)]],
                    rows.at[a * _NBUF + b], gsem[a]).wait()
            for b in range(_NBUF):
                w = q * _NBUF + b
                pltpu.async_copy(rows.at[a * _NBUF + b], acc.at[rall.at[w]],
                                 ssem[a], add=True)
            for b in range(_NBUF):
                w = q * _NBUF + b
                pltpu.make_async_copy(rows.at[a * _NBUF + b],
                                      acc.at[rall.at[w]], ssem[a]).wait()

        nq = nwin // _NBUF
        assert nq % 2 == 1
        issue_group(0, 0)

        @pl.loop(0, nq - 1, step=2)
        def _(q):
            issue_group(1, q + 1)
            process_group(0, q)
            issue_group(0, q + 2)
            process_group(1, q + 1)

        process_group(0, nq - 1)

        plsc.subcore_barrier()

        @pl.when(sid < 15)
        def _():
            pltpu.sync_copy(acc.at[pl.ds(sid * rps, rps)],
                            out_hbm.at[cid, pl.ds(sid * rps, rps)])

        @pl.when(sid == 15)
        def _():
            pltpu.sync_copy(acc.at[pl.ds(15 * rps, last_rows)],
                            out_hbm.at[cid, pl.ds(15 * rps, last_rows)])

    return agg_kernel(feat2, send, recv.reshape(16, nwin, _W))


_BITONIC_STAGES = [(2, 1), (4, 2), (4, 1), (8, 4), (8, 2), (8, 1),
                   (16, 8), (16, 4), (16, 2), (16, 1)]


def _dense_body(part_ref, oh_ref, A1_ref, cb1t_ref, A2_ref, cb2t_ref,
                A3_ref, lb_ref, W1Ta_ref, W1Tb_ref,
                b1r_ref, gam_ref, bet_ref, W2T_ref, b2r_ref,
                out_h_ref, out_oh_ref, h_scr, st_scr, *, bsz, n, dx, l):
    p = pl.program_id(0)
    j = pl.program_id(1)
    hi = lax.Precision.HIGHEST

    agg_x = jnp.concatenate([part_ref[0], part_ref[1][:, :dx - 80]], axis=1)
    new_oh = part_ref[1][:, dx - 80:dx - 80 + l] + oh_ref[...]
    out_oh_ref[...] = new_oh

    @pl.when(p == 0)
    def _phase0():
        vp = new_oh
        li = lax.broadcasted_iota(jnp.int32, (1, l), 1)
        for k, jj in _BITONIC_STAGES:
            zc = jnp.zeros((bsz, jj), jnp.float32)
            up = jnp.concatenate([vp[:, jj:], zc], axis=1)
            dn = jnp.concatenate([zc, vp[:, :l - jj]], axis=1)
            is_lo = (li & jj) == 0
            pv = jnp.where(is_lo, up, dn)
            keep_min = is_lo == ((li & k) == 0)
            vp = jnp.where(keep_min, jnp.minimum(vp, pv),
                           jnp.maximum(vp, pv))
        s = vp

        h1f = jnp.maximum(
            jnp.dot(s, A1_ref[...], precision=hi,
                    preferred_element_type=jnp.float32) + cb1t_ref[...], 0.0)
        h2f = jnp.maximum(
            jnp.dot(h1f, A2_ref[...], precision=hi,
                    preferred_element_type=jnp.float32) + cb2t_ref[...], 0.0)
        res = jnp.dot(h2f, A3_ref[...], precision=hi,
                      preferred_element_type=jnp.float32) + lb_ref[...]

        hb = (jnp.dot(agg_x, W1Ta_ref[...], precision=hi,
                      preferred_element_type=jnp.float32)
              + jnp.dot(res, W1Tb_ref[...], precision=hi,
                        preferred_element_type=jnp.float32)
              + b1r_ref[...])
        h_scr[pl.ds(j * bsz, bsz), :] = hb

        @pl.when(j == 0)
        def _():
            st_scr[...] = jnp.zeros_like(st_scr)

        st_scr[0:1, :] += jnp.sum(hb, axis=0, keepdims=True)
        st_scr[1:2, :] += jnp.sum(hb * hb, axis=0, keepdims=True)
        out_h_ref[...] = hb

    @pl.when(p == 1)
    def _phase1():
        mu = st_scr[0:1, :] / n
        var = st_scr[1:2, :] / n - mu * mu
        rs = lax.rsqrt(var + 1e-5)
        hb = h_scr[pl.ds(j * bsz, bsz), :]
        hn = jnp.maximum((hb - mu) * rs * gam_ref[...] + bet_ref[...], 0.0)
        out_h_ref[...] = (jnp.dot(hn, W2T_ref[...], precision=hi,
                                  preferred_element_type=jnp.float32)
                          + b2r_ref[...])


def _dense_call(partial, onehots, A1, cb1t, A2, cb2t, A3, lb,
                W1Ta, W1Tb, b1r, gam, bet, W2T, b2r, interpret=False):
    n, l = onehots.shape
    dx = 128
    dc = partial.shape[2]
    bsz = 1000
    nb = n // bsz

    def full(a):
        return pl.BlockSpec(a.shape, lambda p, j: (0,) * a.ndim)

    body = functools.partial(_dense_body, bsz=bsz, n=n, dx=dx, l=l)
    return pl.pallas_call(
        body,
        grid=(2, nb),
        in_specs=[
            pl.BlockSpec((2, bsz, dc), lambda p, j: (0, j, 0)),
            pl.BlockSpec((bsz, l), lambda p, j: (j, 0)),
            full(A1), full(cb1t), full(A2), full(cb2t), full(A3),
            full(lb), full(W1Ta), full(W1Tb),
            full(b1r), full(gam), full(bet), full(W2T), full(b2r),
        ],
        out_specs=[
            pl.BlockSpec((bsz, dx), lambda p, j: (p * j, 0)),
            pl.BlockSpec((bsz, l), lambda p, j: (j, 0)),
        ],
        out_shape=[
            jax.ShapeDtypeStruct((n, dx), jnp.float32),
            jax.ShapeDtypeStruct((n, l), jnp.float32),
        ],
        scratch_shapes=[
            pltpu.VMEM((n, dx), jnp.float32),
            pltpu.VMEM((8, dx), jnp.float32),
        ],
        compiler_params=pltpu.CompilerParams(
            dimension_semantics=("arbitrary", "arbitrary")),
        interpret=interpret,
    )(partial, onehots, A1, cb1t, A2, cb2t, A3, lb,
      W1Ta, W1Tb, b1r, gam, bet, W2T, b2r)


def kernel(x, onehots, edge_index, batch_sample_indices, n_sample_nodes, adjs,
           conv1_w, conv1_b, conv2_w, conv2_b, lin16_w, lin16_b,
           W1, b1, bn_gamma, bn_beta, W2, b2):
    n, dx = x.shape
    l = onehots.shape[1]

    fa = x[:, :80]
    fb = jnp.concatenate(
        [x[:, 80:], onehots, jnp.zeros((n, 160 - dx - l), jnp.float32)],
        axis=1)
    feat2 = jnp.concatenate([fa, fb], axis=0)
    send = edge_index[0]
    recv = edge_index[1]
    partial = _sc_aggregate(feat2, send, recv, n)

    eyes = [jnp.eye(l, k=1 - k, dtype=jnp.float32) for k in range(3)]
    A1 = sum(eyes[k][:, :, None] * conv1_w[:, 0, k][None, None, :]
             for k in range(3)).reshape(l, l * 8)
    cb1t = jnp.tile(conv1_b, (l,)).reshape(1, l * 8)
    A2 = sum(eyes[k][:, None, :, None]
             * jnp.transpose(conv2_w[:, :, k])[None, :, None, :]
             for k in range(3)).reshape(l * 8, l * 16)
    cb2t = jnp.tile(conv2_b, (l,)).reshape(1, l * 16)
    A3 = jnp.tile(lin16_w.T / l, (l, 1))
    lb = lin16_b.reshape(1, 8)
    W1Ta = W1[:, :dx].T
    W1Tb = W1[:, dx:].T
    b1r = b1.reshape(1, dx)
    gam = bn_gamma.reshape(1, dx)
    bet = bn_beta.reshape(1, dx)
    W2T = W2.T
    b2r = b2.reshape(1, dx)

    h, new_oh = _dense_call(partial, onehots, A1, cb1t, A2, cb2t, A3, lb,
                            W1Ta, W1Tb, b1r, gam, bet, W2T, b2r)
    return (h, new_oh)


# speedup vs baseline: 10.1579x; 1.0787x over previous
import functools

import jax
import jax.numpy as jnp
from jax import lax
from jax.experimental import pallas as pl
from jax.experimental.pallas import tpu as pltpu
from jax.experimental.pallas import tpu_sc as plsc

_W = 80
_NBUF = 2


def _sc_aggregate(feat2, send, recv, n):
    dc = feat2.shape[1]
    e = send.shape[0]
    epw = e // 16
    nwin = epw // _W
    rps = 640
    npad = 16 * rps
    zr = 128
    nz = rps // zr
    last_rows = n - 15 * rps

    mesh = plsc.VectorSubcoreMesh(core_axis_name="c", subcore_axis_name="s")

    @functools.partial(
        pl.kernel,
        out_type=jax.ShapeDtypeStruct((2, n, dc), jnp.float32),
        mesh=mesh,
        scratch_types=[
            pltpu.VMEM((epw,), jnp.int32),
            pltpu.VMEM((nwin, _W), jnp.int32),
            pltpu.VMEM((2 * _NBUF, _W, dc), jnp.float32),
            pltpu.VMEM((zr, dc), jnp.float32),
            pltpu.VMEM_SHARED((npad, dc), jnp.float32),
        ] + [pltpu.SemaphoreType.DMA] * 4,
        compiler_params=pltpu.CompilerParams(use_tc_tiling_on_sc=False),
    )
    def agg_kernel(feat_hbm, send_hbm, recv_hbm, out_hbm,
                   sall, rall, rows, zbuf, acc, gsem0, gsem1, ssem0, ssem1):
        gsem = (gsem0, gsem1)
        ssem = (ssem0, ssem1)
        cid = lax.axis_index("c")
        sid = lax.axis_index("s")

        @pl.loop(0, zr)
        def _(r):
            @pl.loop(0, dc, step=16)
            def _(c0):
                zbuf[r, pl.ds(c0, 16)] = jnp.zeros((16,), jnp.float32)

        @pl.loop(0, nz)
        def _(j):
            pltpu.sync_copy(zbuf, acc.at[pl.ds(sid * rps + j * zr, zr)])

        plsc.subcore_barrier()

        base = sid * epw
        pltpu.sync_copy(send_hbm.at[pl.ds(base, epw)], sall)
        pltpu.sync_copy(recv_hbm.at[sid], rall)

        @pl.when(cid == 1)
        def _():
            @pl.loop(0, epw, step=16)
            def _(i):
                sall[pl.ds(i, 16)] = sall[pl.ds(i, 16)] + n

        def issue_group(a, q):
            for b in range(_NBUF):
                w = q * _NBUF + b
                pltpu.async_copy(
                    feat_hbm.at[sall.at[pl.ds(w * _W, _W)]],
                    rows.at[a * _NBUF + b], gsem[a])

        def process_group(a, q):
            for b in range(_NBUF):
                w = q * _NBUF + b
                pltpu.make_async_copy(
                    feat_hbm.at[sall.at[pl.ds(w * _W, _W)]],
                    rows.at[a * _NBUF + b], gsem[a]).wait()
            for b in range(_NBUF):
                w = q * _NBUF + b
                pltpu.async_copy(rows.at[a * _NBUF + b], acc.at[rall.at[w]],
                                 ssem[a], add=True)
            for b in range(_NBUF):
                w = q * _NBUF + b
                pltpu.make_async_copy(rows.at[a * _NBUF + b],
                                      acc.at[rall.at[w]], ssem[a]).wait()

        nq = nwin // _NBUF
        assert nq % 2 == 1
        issue_group(0, 0)

        @pl.loop(0, nq - 1, step=2)
        def _(q):
            issue_group(1, q + 1)
            process_group(0, q)
            issue_group(0, q + 2)
            process_group(1, q + 1)

        process_group(0, nq - 1)

        plsc.subcore_barrier()

        @pl.when(sid < 15)
        def _():
            pltpu.sync_copy(acc.at[pl.ds(sid * rps, rps)],
                            out_hbm.at[cid, pl.ds(sid * rps, rps)])

        @pl.when(sid == 15)
        def _():
            pltpu.sync_copy(acc.at[pl.ds(15 * rps, last_rows)],
                            out_hbm.at[cid, pl.ds(15 * rps, last_rows)])

    return agg_kernel(feat2, send, recv.reshape(16, nwin, _W))


_BITONIC_STAGES = [(2, 1), (4, 2), (4, 1), (8, 4), (8, 2), (8, 1),
                   (16, 8), (16, 4), (16, 2), (16, 1)]


def _dense_body(part_ref, oh_ref, ohp_ref, ppp_ref,
                A1_ref, cb1t_ref, A2_ref, cb2t_ref,
                A3_ref, lb_ref, W1Ta_ref, W1Tb_ref,
                b1r_ref, gam_ref, bet_ref, W2T_ref, b2r_ref,
                out_h_ref, out_oh_ref, *, n, dx, l):
    hi = lax.Precision.HIGHEST

    agg_x = jnp.concatenate([part_ref[0], part_ref[1][:, :dx - 80]], axis=1)
    new_oh = part_ref[1][:, dx - 80:dx - 80 + l] + oh_ref[...]
    out_oh_ref[...] = new_oh

    vp = ohp_ref[...] + ppp_ref[...]
    li = lax.broadcasted_iota(jnp.int32, (1, 128), 1) & 15
    for k, jj in _BITONIC_STAGES:
        zc = jnp.zeros((n // 8, jj), jnp.float32)
        up = jnp.concatenate([vp[:, jj:], zc], axis=1)
        dn = jnp.concatenate([zc, vp[:, :128 - jj]], axis=1)
        is_lo = (li & jj) == 0
        pv = jnp.where(is_lo, up, dn)
        keep_min = is_lo == ((li & k) == 0)
        vp = jnp.where(keep_min, jnp.minimum(vp, pv),
                       jnp.maximum(vp, pv))
    s = jnp.concatenate(
        [vp[:, None, j * l:(j + 1) * l] for j in range(8)],
        axis=1).reshape(n, l)

    h1f = jnp.maximum(
        jnp.dot(s, A1_ref[...], precision=hi,
                preferred_element_type=jnp.float32) + cb1t_ref[...], 0.0)
    h2f = jnp.maximum(
        jnp.dot(h1f, A2_ref[...], precision=hi,
                preferred_element_type=jnp.float32) + cb2t_ref[...], 0.0)
    res = jnp.dot(h2f, A3_ref[...], precision=hi,
                  preferred_element_type=jnp.float32) + lb_ref[...]

    hb = (jnp.dot(agg_x, W1Ta_ref[...], precision=hi,
                  preferred_element_type=jnp.float32)
          + jnp.dot(res, W1Tb_ref[...], precision=hi,
                    preferred_element_type=jnp.float32)
          + b1r_ref[...])

    mu = jnp.sum(hb, axis=0, keepdims=True) / n
    var = jnp.sum(hb * hb, axis=0, keepdims=True) / n - mu * mu
    rs = lax.rsqrt(var + 1e-5)
    hn = jnp.maximum((hb - mu) * rs * gam_ref[...] + bet_ref[...], 0.0)
    out_h_ref[...] = (jnp.dot(hn, W2T_ref[...], precision=hi,
                              preferred_element_type=jnp.float32)
                      + b2r_ref[...])


def _dense_call(partial, onehots, A1, cb1t, A2, cb2t, A3, lb,
                W1Ta, W1Tb, b1r, gam, bet, W2T, b2r, interpret=False):
    n, l = onehots.shape
    dx = 128
    ohp = onehots.reshape(n // 8, 8 * l)
    ppp = partial[1, :, dx - 80:dx - 80 + l].reshape(n // 8, 8 * l)

    body = functools.partial(_dense_body, n=n, dx=dx, l=l)
    return pl.pallas_call(
        body,
        out_shape=[
            jax.ShapeDtypeStruct((n, dx), jnp.float32),
            jax.ShapeDtypeStruct((n, l), jnp.float32),
        ],
        interpret=interpret,
    )(partial, onehots, ohp, ppp, A1, cb1t, A2, cb2t, A3, lb,
      W1Ta, W1Tb, b1r, gam, bet, W2T, b2r)


def kernel(x, onehots, edge_index, batch_sample_indices, n_sample_nodes, adjs,
           conv1_w, conv1_b, conv2_w, conv2_b, lin16_w, lin16_b,
           W1, b1, bn_gamma, bn_beta, W2, b2):
    n, dx = x.shape
    l = onehots.shape[1]

    fa = x[:, :80]
    fb = jnp.concatenate(
        [x[:, 80:], onehots, jnp.zeros((n, 160 - dx - l), jnp.float32)],
        axis=1)
    feat2 = jnp.concatenate([fa, fb], axis=0)
    send = edge_index[0]
    recv = edge_index[1]
    partial = _sc_aggregate(feat2, send, recv, n)

    eyes = [jnp.eye(l, k=1 - k, dtype=jnp.float32) for k in range(3)]
    A1 = sum(eyes[k][:, :, None] * conv1_w[:, 0, k][None, None, :]
             for k in range(3)).reshape(l, l * 8)
    cb1t = jnp.tile(conv1_b, (l,)).reshape(1, l * 8)
    A2 = sum(eyes[k][:, None, :, None]
             * jnp.transpose(conv2_w[:, :, k])[None, :, None, :]
             for k in range(3)).reshape(l * 8, l * 16)
    cb2t = jnp.tile(conv2_b, (l,)).reshape(1, l * 16)
    A3 = jnp.tile(lin16_w.T / l, (l, 1))
    lb = lin16_b.reshape(1, 8)
    W1Ta = W1[:, :dx].T
    W1Tb = W1[:, dx:].T
    b1r = b1.reshape(1, dx)
    gam = bn_gamma.reshape(1, dx)
    bet = bn_beta.reshape(1, dx)
    W2T = W2.T
    b2r = b2.reshape(1, dx)

    h, new_oh = _dense_call(partial, onehots, A1, cb1t, A2, cb2t, A3, lb,
                            W1Ta, W1Tb, b1r, gam, bet, W2T, b2r)
    return (h, new_oh)


# speedup vs baseline: 12.3671x; 1.2175x over previous
import functools

import jax
import jax.numpy as jnp
from jax import lax
from jax.experimental import pallas as pl
from jax.experimental.pallas import tpu as pltpu
from jax.experimental.pallas import tpu_sc as plsc

_W = 80
_NBUF = 2


def _sc_aggregate(feat2, send, recv, n):
    dc = feat2.shape[1]
    e = send.shape[0]
    epw = e // 16
    nwin = epw // _W
    rps = 640
    npad = 16 * rps
    zr = 128
    nz = rps // zr
    last_rows = n - 15 * rps

    mesh = plsc.VectorSubcoreMesh(core_axis_name="c", subcore_axis_name="s")

    @functools.partial(
        pl.kernel,
        out_type=jax.ShapeDtypeStruct((2, n, dc), jnp.float32),
        mesh=mesh,
        scratch_types=[
            pltpu.VMEM((epw,), jnp.int32),
            pltpu.VMEM((nwin, _W), jnp.int32),
            pltpu.VMEM((2 * _NBUF, _W, dc), jnp.float32),
            pltpu.VMEM((zr, dc), jnp.float32),
            pltpu.VMEM_SHARED((npad, dc), jnp.float32),
        ] + [pltpu.SemaphoreType.DMA] * 4,
        compiler_params=pltpu.CompilerParams(use_tc_tiling_on_sc=False),
    )
    def agg_kernel(feat_hbm, send_hbm, recv_hbm, out_hbm,
                   sall, rall, rows, zbuf, acc, gsem0, gsem1, ssem0, ssem1):
        gsem = (gsem0, gsem1)
        ssem = (ssem0, ssem1)
        cid = lax.axis_index("c")
        sid = lax.axis_index("s")

        @pl.loop(0, zr)
        def _(r):
            @pl.loop(0, dc, step=16)
            def _(c0):
                zbuf[r, pl.ds(c0, 16)] = jnp.zeros((16,), jnp.float32)

        @pl.loop(0, nz)
        def _(j):
            pltpu.sync_copy(zbuf, acc.at[pl.ds(sid * rps + j * zr, zr)])

        plsc.subcore_barrier()

        base = sid * epw
        pltpu.sync_copy(send_hbm.at[pl.ds(base, epw)], sall)
        pltpu.sync_copy(recv_hbm.at[sid], rall)

        @pl.when(cid == 1)
        def _():
            @pl.loop(0, epw, step=16)
            def _(i):
                sall[pl.ds(i, 16)] = sall[pl.ds(i, 16)] + n

        def issue_group(a, q):
            for b in range(_NBUF):
                w = q * _NBUF + b
                pltpu.async_copy(
                    feat_hbm.at[sall.at[pl.ds(w * _W, _W)]],
                    rows.at[a * _NBUF + b], gsem[a])

        def process_group(a, q):
            for b in range(_NBUF):
                w = q * _NBUF + b
                pltpu.make_async_copy(
                    feat_hbm.at[sall.at[pl.ds(w * _W, _W)]],
                    rows.at[a * _NBUF + b], gsem[a]).wait()
            for b in range(_NBUF):
                w = q * _NBUF + b
                pltpu.async_copy(rows.at[a * _NBUF + b], acc.at[rall.at[w]],
                                 ssem[a], add=True)
            for b in range(_NBUF):
                w = q * _NBUF + b
                pltpu.make_async_copy(rows.at[a * _NBUF + b],
                                      acc.at[rall.at[w]], ssem[a]).wait()

        nq = nwin // _NBUF
        assert nq % 2 == 1
        issue_group(0, 0)

        @pl.loop(0, nq - 1, step=2)
        def _(q):
            issue_group(1, q + 1)
            process_group(0, q)
            issue_group(0, q + 2)
            process_group(1, q + 1)

        process_group(0, nq - 1)

        plsc.subcore_barrier()

        @pl.when(sid < 15)
        def _():
            pltpu.sync_copy(acc.at[pl.ds(sid * rps, rps)],
                            out_hbm.at[cid, pl.ds(sid * rps, rps)])

        @pl.when(sid == 15)
        def _():
            pltpu.sync_copy(acc.at[pl.ds(15 * rps, last_rows)],
                            out_hbm.at[cid, pl.ds(15 * rps, last_rows)])

    return agg_kernel(feat2, send, recv.reshape(16, nwin, _W))


_BITONIC_STAGES = [(2, 1), (4, 2), (4, 1), (8, 4), (8, 2), (8, 1),
                   (16, 8), (16, 4), (16, 2), (16, 1)]


def _dense_body(part_ref, oh_ref, ohp_ref, ppp_ref,
                A1_ref, cb1t_ref, A2_ref, cb2t_ref,
                A3_ref, lb_ref, W1Ta_ref, W1Tb_ref,
                b1r_ref, gam_ref, bet_ref, W2T_ref, b2r_ref,
                out_h_ref, out_oh_ref, *, n, dx, l):

    agg_x = jnp.concatenate([part_ref[0], part_ref[1][:, :dx - 80]], axis=1)
    new_oh = part_ref[1][:, dx - 80:dx - 80 + l] + oh_ref[...]
    out_oh_ref[...] = new_oh

    vp = ohp_ref[...] + ppp_ref[...]
    li = lax.broadcasted_iota(jnp.int32, (1, 128), 1) & 15
    for k, jj in _BITONIC_STAGES:
        zc = jnp.zeros((n // 8, jj), jnp.float32)
        up = jnp.concatenate([vp[:, jj:], zc], axis=1)
        dn = jnp.concatenate([zc, vp[:, :128 - jj]], axis=1)
        is_lo = (li & jj) == 0
        pv = jnp.where(is_lo, up, dn)
        keep_min = is_lo == ((li & k) == 0)
        vp = jnp.where(keep_min, jnp.minimum(vp, pv),
                       jnp.maximum(vp, pv))
    s = jnp.concatenate(
        [vp[:, None, j * l:(j + 1) * l] for j in range(8)],
        axis=1).reshape(n, l)

    h1f = jnp.maximum(
        jnp.dot(s, A1_ref[...],
                preferred_element_type=jnp.float32) + cb1t_ref[...], 0.0)
    h2f = jnp.maximum(
        jnp.dot(h1f, A2_ref[...],
                preferred_element_type=jnp.float32) + cb2t_ref[...], 0.0)
    res = jnp.dot(h2f, A3_ref[...],
                  preferred_element_type=jnp.float32) + lb_ref[...]

    hb = (jnp.dot(agg_x, W1Ta_ref[...],
                  preferred_element_type=jnp.float32)
          + jnp.dot(res, W1Tb_ref[...],
                    preferred_element_type=jnp.float32)
          + b1r_ref[...])

    mu = jnp.sum(hb, axis=0, keepdims=True) / n
    var = jnp.sum(hb * hb, axis=0, keepdims=True) / n - mu * mu
    rs = lax.rsqrt(var + 1e-5)
    hn = jnp.maximum((hb - mu) * rs * gam_ref[...] + bet_ref[...], 0.0)
    out_h_ref[...] = (jnp.dot(hn, W2T_ref[...],
                              preferred_element_type=jnp.float32)
                      + b2r_ref[...])


def _dense_call(partial, onehots, A1, cb1t, A2, cb2t, A3, lb,
                W1Ta, W1Tb, b1r, gam, bet, W2T, b2r, interpret=False):
    n, l = onehots.shape
    dx = 128
    ohp = onehots.reshape(n // 8, 8 * l)
    ppp = partial[1, :, dx - 80:dx - 80 + l].reshape(n // 8, 8 * l)

    body = functools.partial(_dense_body, n=n, dx=dx, l=l)
    return pl.pallas_call(
        body,
        out_shape=[
            jax.ShapeDtypeStruct((n, dx), jnp.float32),
            jax.ShapeDtypeStruct((n, l), jnp.float32),
        ],
        interpret=interpret,
    )(partial, onehots, ohp, ppp, A1, cb1t, A2, cb2t, A3, lb,
      W1Ta, W1Tb, b1r, gam, bet, W2T, b2r)


def kernel(x, onehots, edge_index, batch_sample_indices, n_sample_nodes, adjs,
           conv1_w, conv1_b, conv2_w, conv2_b, lin16_w, lin16_b,
           W1, b1, bn_gamma, bn_beta, W2, b2):
    n, dx = x.shape
    l = onehots.shape[1]

    fa = x[:, :80]
    fb = jnp.concatenate(
        [x[:, 80:], onehots, jnp.zeros((n, 160 - dx - l), jnp.float32)],
        axis=1)
    feat2 = jnp.concatenate([fa, fb], axis=0)
    send = edge_index[0]
    recv = edge_index[1]
    partial = _sc_aggregate(feat2, send, recv, n)

    eyes = [jnp.eye(l, k=1 - k, dtype=jnp.float32) for k in range(3)]
    A1 = sum(eyes[k][:, :, None] * conv1_w[:, 0, k][None, None, :]
             for k in range(3)).reshape(l, l * 8)
    cb1t = jnp.tile(conv1_b, (l,)).reshape(1, l * 8)
    A2 = sum(eyes[k][:, None, :, None]
             * jnp.transpose(conv2_w[:, :, k])[None, :, None, :]
             for k in range(3)).reshape(l * 8, l * 16)
    cb2t = jnp.tile(conv2_b, (l,)).reshape(1, l * 16)
    A3 = jnp.tile(lin16_w.T / l, (l, 1))
    lb = lin16_b.reshape(1, 8)
    W1Ta = W1[:, :dx].T
    W1Tb = W1[:, dx:].T
    b1r = b1.reshape(1, dx)
    gam = bn_gamma.reshape(1, dx)
    bet = bn_beta.reshape(1, dx)
    W2T = W2.T
    b2r = b2.reshape(1, dx)

    h, new_oh = _dense_call(partial, onehots, A1, cb1t, A2, cb2t, A3, lb,
                            W1Ta, W1Tb, b1r, gam, bet, W2T, b2r)
    return (h, new_oh)
